# Initial kernel scaffold; baseline (speedup 1.0000x reference)
import jax
import jax.numpy as jnp
from jax.experimental import pallas as pl


def kernel(x, edge_index, W1, b1, W2, b2):
    raise NotImplementedError("write your pallas kernel here")


# speedup vs baseline: 15.5050x; 15.5050x over previous
import functools

import jax
import jax.numpy as jnp
from jax import lax
from jax.experimental import pallas as pl
from jax.experimental.pallas import tpu as pltpu
from jax.experimental.pallas import tpu_sc as plsc

CHUNK = 128
N_TILES = 32
SUBCORES = 16


def _make_deg_kernel(n_pad, n_chunks):
  chunks_per_tile = n_chunks // N_TILES
  rows_per_tile = n_pad // SUBCORES
  mesh = plsc.VectorSubcoreMesh(core_axis_name="c", subcore_axis_name="s")

  @functools.partial(
      pl.kernel,
      out_type=jax.ShapeDtypeStruct((2 * n_pad,), jnp.float32),
      mesh=mesh,
      scratch_types=[
          pltpu.VMEM_SHARED((n_pad,), jnp.float32),
          pltpu.VMEM((chunks_per_tile, CHUNK), jnp.int32),
          pltpu.VMEM((CHUNK,), jnp.float32),
          pltpu.VMEM((rows_per_tile,), jnp.float32),
      ],
  )
  def deg_kernel(dst_hbm, out_hbm, acc, dstv, ones_v, zeros_v):
    c = lax.axis_index("c")
    s = lax.axis_index("s")
    tile = c * SUBCORES + s
    base = s * rows_per_tile

    for i in range(CHUNK // 16):
      ones_v[pl.ds(i * 16, 16)] = jnp.full((16,), 1.0, jnp.float32)
    for i in range(rows_per_tile // 16):
      zeros_v[pl.ds(i * 16, 16)] = jnp.zeros((16,), jnp.float32)

    pltpu.sync_copy(dst_hbm.at[pl.ds(tile * chunks_per_tile, chunks_per_tile)], dstv)
    pltpu.sync_copy(zeros_v, acc.at[pl.ds(base, rows_per_tile)])
    plsc.subcore_barrier()

    @pl.loop(0, chunks_per_tile)
    def _(j):
      pltpu.sync_copy(ones_v, acc.at[dstv.at[j]], add=True)

    plsc.subcore_barrier()
    pltpu.sync_copy(acc.at[pl.ds(base, rows_per_tile)], zeros_v)
    pltpu.sync_copy(zeros_v, out_hbm.at[pl.ds(c * n_pad + base, rows_per_tile)])

  return deg_kernel


def _make_agg_kernel(n_pad, w, n_chunks):
  chunks_per_tile = n_chunks // N_TILES
  rows_per_tile = n_pad // SUBCORES
  mesh = plsc.VectorSubcoreMesh(core_axis_name="c", subcore_axis_name="s")

  @functools.partial(
      pl.kernel,
      out_type=jax.ShapeDtypeStruct((2 * n_pad, w), jnp.float32),
      mesh=mesh,
      scratch_types=[
          pltpu.VMEM_SHARED((n_pad, w), jnp.float32),
          pltpu.VMEM((chunks_per_tile, CHUNK), jnp.int32),
          pltpu.VMEM((chunks_per_tile, CHUNK), jnp.int32),
          pltpu.VMEM((CHUNK, w), jnp.float32),
      ],
      compiler_params=pltpu.CompilerParams(use_tc_tiling_on_sc=False),
  )
  def agg_kernel(hs_hbm, zeros_hbm, src_hbm, dst_hbm, out_hbm,
                 acc, srcv, dstv, rows):
    c = lax.axis_index("c")
    s = lax.axis_index("s")
    tile = c * SUBCORES + s
    base = s * rows_per_tile

    pltpu.sync_copy(src_hbm.at[pl.ds(tile * chunks_per_tile, chunks_per_tile)], srcv)
    pltpu.sync_copy(dst_hbm.at[pl.ds(tile * chunks_per_tile, chunks_per_tile)], dstv)

    @pl.when(c == 0)
    def _():
      pltpu.sync_copy(hs_hbm.at[pl.ds(base, rows_per_tile)],
                      acc.at[pl.ds(base, rows_per_tile)])

    @pl.when(c != 0)
    def _():
      pltpu.sync_copy(zeros_hbm.at[pl.ds(base, rows_per_tile)],
                      acc.at[pl.ds(base, rows_per_tile)])

    plsc.subcore_barrier()

    @pl.loop(0, chunks_per_tile)
    def _(j):
      pltpu.sync_copy(hs_hbm.at[srcv.at[j]], rows)
      pltpu.sync_copy(rows, acc.at[dstv.at[j]], add=True)

    plsc.subcore_barrier()
    pltpu.sync_copy(acc.at[pl.ds(base, rows_per_tile)],
                    out_hbm.at[pl.ds(c * n_pad + base, rows_per_tile)])

  return agg_kernel


def _tc_pre_body(dp_ref, x_ref, w1_ref, dinv_ref, hs_ref):
  deg = dp_ref[0] + dp_ref[1] + 1.0
  dinv = lax.rsqrt(deg)
  h = jnp.dot(x_ref[...], w1_ref[...], preferred_element_type=jnp.float32)
  dinv_ref[...] = dinv
  hs_ref[...] = h * dinv


def _tc_mid_body(aggp_ref, dinv_ref, b1_ref, w2_ref, hs2_ref):
  agg = aggp_ref[0] + aggp_ref[1]
  dinv = dinv_ref[...]
  h1 = jnp.maximum(agg * dinv + b1_ref[...], 0.0)
  hs2_ref[...] = jnp.dot(h1, w2_ref[...], preferred_element_type=jnp.float32) * dinv


def _tc_post_body(aggp_ref, dinv_ref, b2_ref, z_ref):
  agg = aggp_ref[0] + aggp_ref[1]
  z_ref[...] = agg * dinv_ref[...] + b2_ref[...]


@jax.jit
def kernel(x, edge_index, W1, b1, W2, b2):
  n, in_dim = x.shape
  hidden = W1.shape[1]
  out_dim = W2.shape[1]
  e = edge_index.shape[1]

  blk = 1024
  n_pad = ((n + blk - 1) // blk + 1) * blk
  grid = n_pad // blk
  per_tile = -(-e // (N_TILES * CHUNK * 8)) * (CHUNK * 8)
  e_pad = per_tile * N_TILES
  n_chunks = e_pad // CHUNK

  x_p = jnp.pad(x, ((0, n_pad - n), (0, 0)))
  fill = jnp.full((e_pad - e,), n, jnp.int32)
  src = jnp.concatenate([edge_index[0], fill]).reshape(n_chunks, CHUNK)
  dst = jnp.concatenate([edge_index[1], fill]).reshape(n_chunks, CHUNK)
  zeros_h = jnp.zeros((n_pad, hidden), jnp.float32)
  zeros_o = jnp.zeros((n_pad, out_dim), jnp.float32)

  deg_partial = _make_deg_kernel(n_pad, n_chunks)(dst)
  dp = deg_partial.reshape(2, n_pad, 1)

  dinv, hs1 = pl.pallas_call(
      _tc_pre_body,
      grid=(grid,),
      in_specs=[
          pl.BlockSpec((2, blk, 1), lambda i: (0, i, 0)),
          pl.BlockSpec((blk, in_dim), lambda i: (i, 0)),
          pl.BlockSpec((in_dim, hidden), lambda i: (0, 0)),
      ],
      out_specs=[
          pl.BlockSpec((blk, 1), lambda i: (i, 0)),
          pl.BlockSpec((blk, hidden), lambda i: (i, 0)),
      ],
      out_shape=[
          jax.ShapeDtypeStruct((n_pad, 1), jnp.float32),
          jax.ShapeDtypeStruct((n_pad, hidden), jnp.float32),
      ],
  )(dp, x_p, W1)

  agg1 = _make_agg_kernel(n_pad, hidden, n_chunks)(hs1, zeros_h, src, dst)
  agg1 = agg1.reshape(2, n_pad, hidden)

  hs2 = pl.pallas_call(
      _tc_mid_body,
      grid=(grid,),
      in_specs=[
          pl.BlockSpec((2, blk, hidden), lambda i: (0, i, 0)),
          pl.BlockSpec((blk, 1), lambda i: (i, 0)),
          pl.BlockSpec((1, hidden), lambda i: (0, 0)),
          pl.BlockSpec((hidden, out_dim), lambda i: (0, 0)),
      ],
      out_specs=pl.BlockSpec((blk, out_dim), lambda i: (i, 0)),
      out_shape=jax.ShapeDtypeStruct((n_pad, out_dim), jnp.float32),
  )(agg1, dinv, b1.reshape(1, hidden), W2)

  agg2 = _make_agg_kernel(n_pad, out_dim, n_chunks)(hs2, zeros_o, src, dst)
  agg2 = agg2.reshape(2, n_pad, out_dim)

  z = pl.pallas_call(
      _tc_post_body,
      grid=(grid,),
      in_specs=[
          pl.BlockSpec((2, blk, out_dim), lambda i: (0, i, 0)),
          pl.BlockSpec((blk, 1), lambda i: (i, 0)),
          pl.BlockSpec((1, out_dim), lambda i: (0, 0)),
      ],
      out_specs=pl.BlockSpec((blk, out_dim), lambda i: (i, 0)),
      out_shape=jax.ShapeDtypeStruct((n_pad, out_dim), jnp.float32),
  )(agg2, dinv, b2.reshape(1, out_dim))

  return z[:n]


# speedup vs baseline: 18.1928x; 1.1734x over previous
import functools

import jax
import jax.numpy as jnp
from jax import lax
from jax.experimental import pallas as pl
from jax.experimental.pallas import tpu as pltpu
from jax.experimental.pallas import tpu_sc as plsc

CHUNK = 128
NBUF = 8
N_TILES = 32
SUBCORES = 16


def _make_deg_kernel(n_pad, n_chunks):
  chunks_per_tile = n_chunks // N_TILES
  rows_per_tile = n_pad // SUBCORES
  mesh = plsc.VectorSubcoreMesh(core_axis_name="c", subcore_axis_name="s")

  @functools.partial(
      pl.kernel,
      out_type=jax.ShapeDtypeStruct((2 * n_pad,), jnp.float32),
      mesh=mesh,
      scratch_types=[
          pltpu.VMEM_SHARED((n_pad,), jnp.float32),
          pltpu.VMEM((chunks_per_tile, CHUNK), jnp.int32),
          pltpu.VMEM((CHUNK,), jnp.float32),
          pltpu.VMEM((rows_per_tile,), jnp.float32),
          pltpu.SemaphoreType.DMA,
      ],
  )
  def deg_kernel(dst_hbm, out_hbm, acc, dstv, ones_v, zeros_v, ssem):
    c = lax.axis_index("c")
    s = lax.axis_index("s")
    tile = c * SUBCORES + s
    base = s * rows_per_tile

    for i in range(CHUNK // 16):
      ones_v[pl.ds(i * 16, 16)] = jnp.full((16,), 1.0, jnp.float32)
    for i in range(rows_per_tile // 16):
      zeros_v[pl.ds(i * 16, 16)] = jnp.zeros((16,), jnp.float32)

    pltpu.sync_copy(dst_hbm.at[pl.ds(tile * chunks_per_tile, chunks_per_tile)], dstv)
    pltpu.sync_copy(zeros_v, acc.at[pl.ds(base, rows_per_tile)])
    plsc.subcore_barrier()

    @pl.loop(0, chunks_per_tile)
    def _(j):
      pltpu.async_copy(ones_v, acc.at[dstv.at[j]], ssem, add=True)

    @pl.loop(0, chunks_per_tile)
    def _(j):
      pltpu.make_async_copy(ones_v, acc.at[dstv.at[j]], ssem).wait()

    plsc.subcore_barrier()
    pltpu.sync_copy(acc.at[pl.ds(base, rows_per_tile)], zeros_v)
    pltpu.sync_copy(zeros_v, out_hbm.at[pl.ds(c * n_pad + base, rows_per_tile)])

  return deg_kernel


def _make_agg_kernel(n_pad, w, n_chunks):
  chunks_per_tile = n_chunks // N_TILES
  rows_per_tile = n_pad // SUBCORES
  mesh = plsc.VectorSubcoreMesh(core_axis_name="c", subcore_axis_name="s")

  @functools.partial(
      pl.kernel,
      out_type=jax.ShapeDtypeStruct((2 * n_pad, w), jnp.float32),
      mesh=mesh,
      scratch_types=[
          pltpu.VMEM_SHARED((n_pad, w), jnp.float32),
          pltpu.VMEM((chunks_per_tile, CHUNK), jnp.int32),
          pltpu.VMEM((chunks_per_tile, CHUNK), jnp.int32),
          pltpu.VMEM((NBUF, CHUNK, w), jnp.float32),
          pltpu.SemaphoreType.DMA((NBUF,)),
      ],
      compiler_params=pltpu.CompilerParams(use_tc_tiling_on_sc=False),
  )
  def agg_kernel(hs_hbm, zeros_hbm, src_hbm, dst_hbm, out_hbm,
                 acc, srcv, dstv, rows, gsem):
    c = lax.axis_index("c")
    s = lax.axis_index("s")
    tile = c * SUBCORES + s
    base = s * rows_per_tile

    pltpu.sync_copy(src_hbm.at[pl.ds(tile * chunks_per_tile, chunks_per_tile)], srcv)
    pltpu.sync_copy(dst_hbm.at[pl.ds(tile * chunks_per_tile, chunks_per_tile)], dstv)

    @pl.when(c == 0)
    def _():
      pltpu.sync_copy(hs_hbm.at[pl.ds(base, rows_per_tile)],
                      acc.at[pl.ds(base, rows_per_tile)])

    @pl.when(c != 0)
    def _():
      pltpu.sync_copy(zeros_hbm.at[pl.ds(base, rows_per_tile)],
                      acc.at[pl.ds(base, rows_per_tile)])

    plsc.subcore_barrier()

    for b in range(NBUF):
      pltpu.async_copy(hs_hbm.at[srcv.at[b]], rows.at[b], gsem.at[b])

    @pl.loop(0, chunks_per_tile // NBUF)
    def _(g):
      for b in range(NBUF):
        j = g * NBUF + b
        pltpu.make_async_copy(hs_hbm.at[srcv.at[j]], rows.at[b], gsem.at[b]).wait()
        pltpu.sync_copy(rows.at[b], acc.at[dstv.at[j]], add=True)

        @pl.when(j + NBUF < chunks_per_tile)
        def _():
          pltpu.async_copy(hs_hbm.at[srcv.at[j + NBUF]], rows.at[b], gsem.at[b])

    plsc.subcore_barrier()
    pltpu.sync_copy(acc.at[pl.ds(base, rows_per_tile)],
                    out_hbm.at[pl.ds(c * n_pad + base, rows_per_tile)])

  return agg_kernel


def _tc_pre_body(dp_ref, x_ref, w1_ref, dinv_ref, hs_ref):
  deg = dp_ref[0] + dp_ref[1] + 1.0
  dinv = lax.rsqrt(deg)
  h = jnp.dot(x_ref[...], w1_ref[...], preferred_element_type=jnp.float32)
  dinv_ref[...] = dinv
  hs_ref[...] = h * dinv


def _tc_mid_body(aggp_ref, dinv_ref, b1_ref, w2_ref, hs2_ref):
  agg = aggp_ref[0] + aggp_ref[1]
  dinv = dinv_ref[...]
  h1 = jnp.maximum(agg * dinv + b1_ref[...], 0.0)
  hs2_ref[...] = jnp.dot(h1, w2_ref[...], preferred_element_type=jnp.float32) * dinv


def _tc_post_body(aggp_ref, dinv_ref, b2_ref, z_ref):
  agg = aggp_ref[0] + aggp_ref[1]
  z_ref[...] = agg * dinv_ref[...] + b2_ref[...]


@jax.jit
def kernel(x, edge_index, W1, b1, W2, b2):
  n, in_dim = x.shape
  hidden = W1.shape[1]
  out_dim = W2.shape[1]
  e = edge_index.shape[1]

  blk = 1024
  n_pad = ((n + blk - 1) // blk + 1) * blk
  grid = n_pad // blk
  per_tile = -(-e // (N_TILES * CHUNK * 8)) * (CHUNK * 8)
  e_pad = per_tile * N_TILES
  n_chunks = e_pad // CHUNK

  x_p = jnp.pad(x, ((0, n_pad - n), (0, 0)))
  fill = jnp.full((e_pad - e,), n, jnp.int32)
  src = jnp.concatenate([edge_index[0], fill]).reshape(n_chunks, CHUNK)
  dst = jnp.concatenate([edge_index[1], fill]).reshape(n_chunks, CHUNK)
  zeros_h = jnp.zeros((n_pad, hidden), jnp.float32)
  zeros_o = jnp.zeros((n_pad, out_dim), jnp.float32)

  deg_partial = _make_deg_kernel(n_pad, n_chunks)(dst)
  dp = deg_partial.reshape(2, n_pad, 1)

  dinv, hs1 = pl.pallas_call(
      _tc_pre_body,
      grid=(grid,),
      in_specs=[
          pl.BlockSpec((2, blk, 1), lambda i: (0, i, 0)),
          pl.BlockSpec((blk, in_dim), lambda i: (i, 0)),
          pl.BlockSpec((in_dim, hidden), lambda i: (0, 0)),
      ],
      out_specs=[
          pl.BlockSpec((blk, 1), lambda i: (i, 0)),
          pl.BlockSpec((blk, hidden), lambda i: (i, 0)),
      ],
      out_shape=[
          jax.ShapeDtypeStruct((n_pad, 1), jnp.float32),
          jax.ShapeDtypeStruct((n_pad, hidden), jnp.float32),
      ],
  )(dp, x_p, W1)

  agg1 = _make_agg_kernel(n_pad, hidden, n_chunks)(hs1, zeros_h, src, dst)
  agg1 = agg1.reshape(2, n_pad, hidden)

  hs2 = pl.pallas_call(
      _tc_mid_body,
      grid=(grid,),
      in_specs=[
          pl.BlockSpec((2, blk, hidden), lambda i: (0, i, 0)),
          pl.BlockSpec((blk, 1), lambda i: (i, 0)),
          pl.BlockSpec((1, hidden), lambda i: (0, 0)),
          pl.BlockSpec((hidden, out_dim), lambda i: (0, 0)),
      ],
      out_specs=pl.BlockSpec((blk, out_dim), lambda i: (i, 0)),
      out_shape=jax.ShapeDtypeStruct((n_pad, out_dim), jnp.float32),
  )(agg1, dinv, b1.reshape(1, hidden), W2)

  agg2 = _make_agg_kernel(n_pad, out_dim, n_chunks)(hs2, zeros_o, src, dst)
  agg2 = agg2.reshape(2, n_pad, out_dim)

  z = pl.pallas_call(
      _tc_post_body,
      grid=(grid,),
      in_specs=[
          pl.BlockSpec((2, blk, out_dim), lambda i: (0, i, 0)),
          pl.BlockSpec((blk, 1), lambda i: (i, 0)),
          pl.BlockSpec((1, out_dim), lambda i: (0, 0)),
      ],
      out_specs=pl.BlockSpec((blk, out_dim), lambda i: (i, 0)),
      out_shape=jax.ShapeDtypeStruct((n_pad, out_dim), jnp.float32),
  )(agg2, dinv, b2.reshape(1, out_dim))

  return z[:n]


# speedup vs baseline: 46.1524x; 2.5368x over previous
import functools

import jax
import jax.numpy as jnp
from jax import lax
from jax.experimental import pallas as pl
from jax.experimental.pallas import tpu as pltpu
from jax.experimental.pallas import tpu_sc as plsc

CHUNK = 128
NBUF = 8
N_TILES = 32
SUBCORES = 16


def _make_deg_kernel(n_pad, n_chunks):
  chunks_per_tile = n_chunks // N_TILES
  rows_per_tile = n_pad // SUBCORES
  mesh = plsc.VectorSubcoreMesh(core_axis_name="c", subcore_axis_name="s")

  @functools.partial(
      pl.kernel,
      out_type=jax.ShapeDtypeStruct((2 * n_pad,), jnp.float32),
      mesh=mesh,
      scratch_types=[
          pltpu.VMEM_SHARED((n_pad,), jnp.float32),
          pltpu.VMEM((chunks_per_tile, CHUNK), jnp.int32),
          pltpu.VMEM((CHUNK,), jnp.float32),
          pltpu.VMEM((rows_per_tile,), jnp.float32),
          pltpu.SemaphoreType.DMA,
      ],
  )
  def deg_kernel(dst_hbm, out_hbm, acc, dstv, ones_v, zeros_v, ssem):
    c = lax.axis_index("c")
    s = lax.axis_index("s")
    tile = c * SUBCORES + s
    base = s * rows_per_tile

    for i in range(CHUNK // 16):
      ones_v[pl.ds(i * 16, 16)] = jnp.full((16,), 1.0, jnp.float32)
    for i in range(rows_per_tile // 16):
      zeros_v[pl.ds(i * 16, 16)] = jnp.zeros((16,), jnp.float32)

    pltpu.sync_copy(dst_hbm.at[pl.ds(tile * chunks_per_tile, chunks_per_tile)], dstv)
    pltpu.sync_copy(zeros_v, acc.at[pl.ds(base, rows_per_tile)])
    plsc.subcore_barrier()

    @pl.loop(0, chunks_per_tile)
    def _(j):
      pltpu.async_copy(ones_v, acc.at[dstv.at[j]], ssem, add=True)

    @pl.loop(0, chunks_per_tile)
    def _(j):
      pltpu.make_async_copy(ones_v, acc.at[dstv.at[j]], ssem).wait()

    plsc.subcore_barrier()
    pltpu.sync_copy(acc.at[pl.ds(base, rows_per_tile)], zeros_v)
    pltpu.sync_copy(zeros_v, out_hbm.at[pl.ds(c * n_pad + base, rows_per_tile)])

  return deg_kernel


def _make_agg_kernel(n_pad, w, n_chunks):
  chunks_per_tile = n_chunks // N_TILES
  rows_per_tile = n_pad // SUBCORES
  mesh = plsc.VectorSubcoreMesh(core_axis_name="c", subcore_axis_name="s")

  @functools.partial(
      pl.kernel,
      out_type=jax.ShapeDtypeStruct((2 * n_pad, w), jnp.float32),
      mesh=mesh,
      scratch_types=[
          pltpu.VMEM_SHARED((n_pad, w), jnp.float32),
          pltpu.VMEM((chunks_per_tile, CHUNK), jnp.int32),
          pltpu.VMEM((chunks_per_tile, CHUNK), jnp.int32),
          pltpu.VMEM((NBUF, CHUNK, w), jnp.float32),
          pltpu.SemaphoreType.DMA((NBUF,)),
      ],
      compiler_params=pltpu.CompilerParams(use_tc_tiling_on_sc=False),
  )
  def agg_kernel(hs_hbm, zeros_hbm, src_hbm, dst_hbm, out_hbm,
                 acc, srcv, dstv, rows, gsem):
    c = lax.axis_index("c")
    s = lax.axis_index("s")
    tile = c * SUBCORES + s
    base = s * rows_per_tile

    pltpu.sync_copy(src_hbm.at[pl.ds(tile * chunks_per_tile, chunks_per_tile)], srcv)
    pltpu.sync_copy(dst_hbm.at[pl.ds(tile * chunks_per_tile, chunks_per_tile)], dstv)

    @pl.when(c == 0)
    def _():
      pltpu.sync_copy(hs_hbm.at[pl.ds(base, rows_per_tile)],
                      acc.at[pl.ds(base, rows_per_tile)])

    @pl.when(c != 0)
    def _():
      pltpu.sync_copy(zeros_hbm.at[pl.ds(base, rows_per_tile)],
                      acc.at[pl.ds(base, rows_per_tile)])

    plsc.subcore_barrier()

    for b in range(NBUF):
      pltpu.async_copy(hs_hbm.at[srcv.at[b]], rows.at[b], gsem.at[b])

    @pl.loop(0, chunks_per_tile // NBUF)
    def _(g):
      for b in range(NBUF):
        j = g * NBUF + b
        pltpu.make_async_copy(hs_hbm.at[srcv.at[j]], rows.at[b], gsem.at[b]).wait()
        pltpu.sync_copy(rows.at[b], acc.at[dstv.at[j]], add=True)

        @pl.when(j + NBUF < chunks_per_tile)
        def _():
          pltpu.async_copy(hs_hbm.at[srcv.at[j + NBUF]], rows.at[b], gsem.at[b])

    plsc.subcore_barrier()
    pltpu.sync_copy(acc.at[pl.ds(base, rows_per_tile)],
                    out_hbm.at[pl.ds(c * n_pad + base, rows_per_tile)])

  return agg_kernel


def _tc_pre_body(dp_ref, x_ref, w1_ref, dinv_ref, hs_ref):
  deg = dp_ref[0] + dp_ref[1] + 1.0
  dinv = lax.rsqrt(deg)
  h = jnp.dot(x_ref[...], w1_ref[...], preferred_element_type=jnp.float32)
  dinv_ref[...] = dinv
  hs_ref[...] = h * dinv


def _tc_mid_body(aggp_ref, dinv_ref, b1_ref, w2_ref, hs2_ref):
  agg = aggp_ref[0] + aggp_ref[1]
  dinv = dinv_ref[...]
  h1 = jnp.maximum(agg * dinv + b1_ref[...], 0.0)
  hs2_ref[...] = jnp.dot(h1, w2_ref[...], preferred_element_type=jnp.float32) * dinv


def _tc_post_body(aggp_ref, dinv_ref, b2_ref, z_ref):
  agg = aggp_ref[0] + aggp_ref[1]
  z_ref[...] = agg * dinv_ref[...] + b2_ref[...]


@jax.jit
def kernel(x, edge_index, W1, b1, W2, b2):
  n, in_dim = x.shape
  hidden = W1.shape[1]
  out_dim = W2.shape[1]
  e = edge_index.shape[1]

  blk = 1024
  n_pad = ((n + blk - 1) // blk + 1) * blk
  grid = n_pad // blk
  per_tile = -(-e // (N_TILES * CHUNK * 8)) * (CHUNK * 8)
  e_pad = per_tile * N_TILES
  n_chunks = e_pad // CHUNK

  x_p = jnp.pad(x, ((0, n_pad - n), (0, 0)))
  fill_idx = n + jnp.arange(e_pad - e, dtype=jnp.int32) % (n_pad - n)
  fill_src = fill_idx
  fill_dst = fill_idx
  src = jnp.concatenate([edge_index[0], fill_src]).reshape(n_chunks, CHUNK)
  dst = jnp.concatenate([edge_index[1], fill_dst]).reshape(n_chunks, CHUNK)
  zeros_h = jnp.zeros((n_pad, hidden), jnp.float32)
  zeros_o = jnp.zeros((n_pad, out_dim), jnp.float32)

  deg_partial = _make_deg_kernel(n_pad, n_chunks)(dst)
  dp = deg_partial.reshape(2, n_pad, 1)

  dinv, hs1 = pl.pallas_call(
      _tc_pre_body,
      grid=(grid,),
      in_specs=[
          pl.BlockSpec((2, blk, 1), lambda i: (0, i, 0)),
          pl.BlockSpec((blk, in_dim), lambda i: (i, 0)),
          pl.BlockSpec((in_dim, hidden), lambda i: (0, 0)),
      ],
      out_specs=[
          pl.BlockSpec((blk, 1), lambda i: (i, 0)),
          pl.BlockSpec((blk, hidden), lambda i: (i, 0)),
      ],
      out_shape=[
          jax.ShapeDtypeStruct((n_pad, 1), jnp.float32),
          jax.ShapeDtypeStruct((n_pad, hidden), jnp.float32),
      ],
  )(dp, x_p, W1)

  agg1 = _make_agg_kernel(n_pad, hidden, n_chunks)(hs1, zeros_h, src, dst)
  agg1 = agg1.reshape(2, n_pad, hidden)

  hs2 = pl.pallas_call(
      _tc_mid_body,
      grid=(grid,),
      in_specs=[
          pl.BlockSpec((2, blk, hidden), lambda i: (0, i, 0)),
          pl.BlockSpec((blk, 1), lambda i: (i, 0)),
          pl.BlockSpec((1, hidden), lambda i: (0, 0)),
          pl.BlockSpec((hidden, out_dim), lambda i: (0, 0)),
      ],
      out_specs=pl.BlockSpec((blk, out_dim), lambda i: (i, 0)),
      out_shape=jax.ShapeDtypeStruct((n_pad, out_dim), jnp.float32),
  )(agg1, dinv, b1.reshape(1, hidden), W2)

  agg2 = _make_agg_kernel(n_pad, out_dim, n_chunks)(hs2, zeros_o, src, dst)
  agg2 = agg2.reshape(2, n_pad, out_dim)

  z = pl.pallas_call(
      _tc_post_body,
      grid=(grid,),
      in_specs=[
          pl.BlockSpec((2, blk, out_dim), lambda i: (0, i, 0)),
          pl.BlockSpec((blk, 1), lambda i: (i, 0)),
          pl.BlockSpec((1, out_dim), lambda i: (0, 0)),
      ],
      out_specs=pl.BlockSpec((blk, out_dim), lambda i: (i, 0)),
      out_shape=jax.ShapeDtypeStruct((n_pad, out_dim), jnp.float32),
  )(agg2, dinv, b2.reshape(1, out_dim))

  return z[:n]
